# Initial kernel scaffold; baseline (speedup 1.0000x reference)
import jax
import jax.numpy as jnp
from jax.experimental import pallas as pl


def kernel(x, edge_index, edge_attr, W1, b1, W2, b2, W3, b3, Wf, bf, We1, be1, We2, be2, Wa1, ba1, Wa2, ba2, Wc1, bc1, Wc2, bc2, Wc3, bc3):
    raise NotImplementedError("write your pallas kernel here")


# speedup vs baseline: 7.1775x; 7.1775x over previous
import functools

import jax
import jax.numpy as jnp
from jax import lax
from jax.experimental import pallas as pl
from jax.experimental.pallas import tpu as pltpu
from jax.experimental.pallas import tpu_sc as plsc

N = 10000
NP = 10240
E = 320000
H = 128
NC, NS = 2, 16
NW = NC * NS
C = 128
NCH = E // C
BN = 1024
BE = 2000

_mesh = plsc.VectorSubcoreMesh(
    core_axis_name="c", subcore_axis_name="s", num_cores=NC, num_subcores=NS)
_sc_params = pltpu.CompilerParams(needs_layout_passes=False)


def _wid():
    return lax.axis_index("c") * NS + lax.axis_index("s")


def _nch_w(wid):
    return (NCH - wid + NW - 1) // NW


def _deg_body(dst_hbm, out_hbm, deg_v, idx_v):
    wid = _wid()

    def zero(i, carry):
        deg_v[pl.ds(i * 16, 16)] = jnp.zeros((16,), jnp.float32)
        return carry
    lax.fori_loop(0, NP // 16, zero, 0)

    ones = jnp.ones((16,), jnp.float32)

    def chunk(i, carry):
        ch = wid + i * NW
        pltpu.sync_copy(dst_hbm.at[pl.ds(ch * C, C)], idx_v)

        def sub(j, c2):
            idx = idx_v[pl.ds(j * 16, 16)]
            plsc.addupdate_scatter(deg_v, [idx], ones)
            return c2
        lax.fori_loop(0, C // 16, sub, 0)
        return carry
    lax.fori_loop(0, _nch_w(wid), chunk, 0)
    pltpu.sync_copy(deg_v, out_hbm.at[wid])


_deg_call = pl.kernel(
    _deg_body,
    out_type=jax.ShapeDtypeStruct((NW, NP), jnp.float32),
    mesh=_mesh,
    compiler_params=_sc_params,
    scratch_types=[
        pltpu.VMEM((NP,), jnp.float32),
        pltpu.VMEM((C,), jnp.int32),
    ],
)


def _seg_body(g_hbm, src_hbm, dst_hbm, zero_hbm, p_hbm,
              idx_s, idx_d, rows, acc, sem):
    cid = lax.axis_index("c")
    sid = lax.axis_index("s")
    wid = cid * NS + sid
    rpw = NP // NS

    pltpu.sync_copy(zero_hbm.at[pl.ds(sid * rpw, rpw)],
                    acc.at[pl.ds(sid * rpw, rpw)])
    plsc.subcore_barrier()

    def chunk(i, carry):
        ch = wid + i * NW
        pltpu.sync_copy(src_hbm.at[pl.ds(ch * C, C)], idx_s)
        pltpu.sync_copy(dst_hbm.at[pl.ds(ch * C, C)], idx_d)
        pltpu.async_copy(g_hbm.at[idx_s], rows, sem).wait()
        pltpu.sync_copy(rows, acc.at[idx_d], add=True)
        return carry
    lax.fori_loop(0, _nch_w(wid), chunk, 0)

    plsc.subcore_barrier()
    pltpu.sync_copy(acc.at[pl.ds(sid * rpw, rpw)],
                    p_hbm.at[cid].at[pl.ds(sid * rpw, rpw)])


_seg_call = pl.kernel(
    _seg_body,
    out_type=jax.ShapeDtypeStruct((NC, NP, H), jnp.float32),
    mesh=_mesh,
    compiler_params=_sc_params,
    scratch_types=[
        pltpu.VMEM((C,), jnp.int32),
        pltpu.VMEM((C,), jnp.int32),
        pltpu.VMEM((C, H), jnp.float32),
        pltpu.VMEM_SHARED((NP, H), jnp.float32),
        pltpu.SemaphoreType.DMA,
    ],
)


def _cls_body(a_hbm, b_hbm, row_hbm, col_hbm, s_hbm,
              idx_r, idx_c, rows_a, rows_b, sem):
    wid = _wid()

    def chunk(i, carry):
        ch = wid + i * NW
        pltpu.sync_copy(row_hbm.at[pl.ds(ch * C, C)], idx_r)
        pltpu.sync_copy(col_hbm.at[pl.ds(ch * C, C)], idx_c)
        pltpu.async_copy(a_hbm.at[idx_r], rows_a, sem).wait()
        pltpu.async_copy(b_hbm.at[idx_c], rows_b, sem).wait()

        def add16(t, c2):
            r = t // (H // 16)
            col0 = (t % (H // 16)) * 16
            sl = pl.ds(col0, 16)
            rows_a[r, sl] = rows_a[r, sl] + rows_b[r, sl]
            return c2
        lax.fori_loop(0, C * H // 16, add16, 0)
        pltpu.sync_copy(rows_a, s_hbm.at[pl.ds(ch * C, C)])
        return carry
    lax.fori_loop(0, _nch_w(wid), chunk, 0)


_cls_call = pl.kernel(
    _cls_body,
    out_type=jax.ShapeDtypeStruct((E, H), jnp.float32),
    mesh=_mesh,
    compiler_params=_sc_params,
    scratch_types=[
        pltpu.VMEM((C,), jnp.int32),
        pltpu.VMEM((C,), jnp.int32),
        pltpu.VMEM((C, H), jnp.float32),
        pltpu.VMEM((C, H), jnp.float32),
        pltpu.SemaphoreType.DMA,
    ],
)


def _g1_kernel(degp_ref, x_ref, w_ref, dinv_ref, g_ref):
    deg = jnp.sum(degp_ref[...], axis=0) + 1.0
    dinv = lax.rsqrt(deg)
    dinv_ref[...] = dinv
    g_ref[...] = (x_ref[...] @ w_ref[...]) * dinv[:, None]


def _g1_call(degp, x_pad, W1):
    return pl.pallas_call(
        _g1_kernel,
        grid=(NP // BN,),
        in_specs=[
            pl.BlockSpec((NW, BN), lambda i: (0, i)),
            pl.BlockSpec((BN, H), lambda i: (i, 0)),
            pl.BlockSpec((H, H), lambda i: (0, 0)),
        ],
        out_specs=[
            pl.BlockSpec((BN,), lambda i: (i,)),
            pl.BlockSpec((BN, H), lambda i: (i, 0)),
        ],
        out_shape=[
            jax.ShapeDtypeStruct((NP,), jnp.float32),
            jax.ShapeDtypeStruct((NP, H), jnp.float32),
        ],
    )(degp, x_pad, W1)


def _comb_kernel(p_ref, g_ref, dinv_ref, b_ref, w_ref, out_ref):
    dinv = dinv_ref[...]
    h = jnp.maximum(
        (p_ref[0] + p_ref[1] + g_ref[...]) * dinv[:, None] + b_ref[...], 0.0)
    out_ref[...] = (h @ w_ref[...]) * dinv[:, None]


def _comb_call(p, g, dinv, b_row, W_next):
    return pl.pallas_call(
        _comb_kernel,
        grid=(NP // BN,),
        in_specs=[
            pl.BlockSpec((NC, BN, H), lambda i: (0, i, 0)),
            pl.BlockSpec((BN, H), lambda i: (i, 0)),
            pl.BlockSpec((BN,), lambda i: (i,)),
            pl.BlockSpec((1, H), lambda i: (0, 0)),
            pl.BlockSpec((H, H), lambda i: (0, 0)),
        ],
        out_specs=pl.BlockSpec((BN, H), lambda i: (i, 0)),
        out_shape=jax.ShapeDtypeStruct((NP, H), jnp.float32),
    )(p, g, dinv, b_row, W_next)


def _post_kernel(p_ref, g_ref, dinv_ref, bf_ref, wa1_ref, ba1_ref,
                 wa2_ref, ba2_ref, w1a_ref, w1b_ref, a_ref, b_out_ref):
    dinv = dinv_ref[...]
    h = jnp.maximum(
        (p_ref[0] + p_ref[1] + g_ref[...]) * dinv[:, None] + bf_ref[...], 0.0)
    t = jnp.maximum(h @ wa1_ref[...] + ba1_ref[...], 0.0)
    att = jax.nn.sigmoid(
        jnp.sum(t * wa2_ref[...], axis=1, keepdims=True) + ba2_ref[...])
    hg = h * att
    a_ref[...] = hg @ w1a_ref[...]
    b_out_ref[...] = hg @ w1b_ref[...]


def _post_call(p, g, dinv, bf_row, Wa1, ba1_row, wa2_row, ba2_11, W1a, W1b):
    return pl.pallas_call(
        _post_kernel,
        grid=(NP // BN,),
        in_specs=[
            pl.BlockSpec((NC, BN, H), lambda i: (0, i, 0)),
            pl.BlockSpec((BN, H), lambda i: (i, 0)),
            pl.BlockSpec((BN,), lambda i: (i,)),
            pl.BlockSpec((1, H), lambda i: (0, 0)),
            pl.BlockSpec((H, H // 2), lambda i: (0, 0)),
            pl.BlockSpec((1, H // 2), lambda i: (0, 0)),
            pl.BlockSpec((1, H // 2), lambda i: (0, 0)),
            pl.BlockSpec((1, 1), lambda i: (0, 0)),
            pl.BlockSpec((H, H), lambda i: (0, 0)),
            pl.BlockSpec((H, H), lambda i: (0, 0)),
        ],
        out_specs=[
            pl.BlockSpec((BN, H), lambda i: (i, 0)),
            pl.BlockSpec((BN, H), lambda i: (i, 0)),
        ],
        out_shape=[
            jax.ShapeDtypeStruct((NP, H), jnp.float32),
            jax.ShapeDtypeStruct((NP, H), jnp.float32),
        ],
    )(p, g, dinv, bf_row, Wa1, ba1_row, wa2_row, ba2_11, W1a, W1b)


def _edge_kernel(s_ref, ea_ref, we1_ref, be1_ref, we2_ref, be2_ref,
                 wc1c_ref, bc1_ref, wc2_ref, bc2_ref, wc3_ref, bc3_ref,
                 out_ref):
    t = jnp.maximum(ea_ref[...] @ we1_ref[...] + be1_ref[...], 0.0)
    wprime = we2_ref[...] @ wc1c_ref[...]
    cprime = be2_ref[...] @ wc1c_ref[...] + bc1_ref[...]
    z = jnp.maximum(t @ wprime + s_ref[...] + cprime, 0.0)
    z2 = jnp.maximum(z @ wc2_ref[...] + bc2_ref[...], 0.0)
    logits = z2 @ wc3_ref[...] + bc3_ref[...]
    m = jnp.max(logits, axis=1, keepdims=True)
    lse = m + jnp.log(jnp.sum(jnp.exp(logits - m), axis=1, keepdims=True))
    out_ref[...] = logits - lse


def _edge_call(S, ea_pad, We1p, be1_row, We2, be2_row, Wc1c, bc1_row,
               Wc2, bc2_row, Wc3, bc3_row):
    full = lambda shape: pl.BlockSpec(shape, lambda i: tuple(0 for _ in shape))
    return pl.pallas_call(
        _edge_kernel,
        grid=(E // BE,),
        in_specs=[
            pl.BlockSpec((BE, H), lambda i: (i, 0)),
            pl.BlockSpec((BE, 8), lambda i: (i, 0)),
            full((8, H)),
            full((1, H)),
            full((H, H)),
            full((1, H)),
            full((H, H)),
            full((1, H)),
            full((H, H // 2)),
            full((1, H // 2)),
            full((H // 2, 2)),
            full((1, 2)),
        ],
        out_specs=pl.BlockSpec((BE, 2), lambda i: (i, 0)),
        out_shape=jax.ShapeDtypeStruct((E, 2), jnp.float32),
    )(S, ea_pad, We1p, be1_row, We2, be2_row, Wc1c, bc1_row,
      Wc2, bc2_row, Wc3, bc3_row)


def kernel(x, edge_index, edge_attr, W1, b1, W2, b2, W3, b3, Wf, bf,
           We1, be1, We2, be2, Wa1, ba1, Wa2, ba2,
           Wc1, bc1, Wc2, bc2, Wc3, bc3):
    src = edge_index[0]
    dst = edge_index[1]
    x_pad = jnp.pad(x, ((0, NP - N), (0, 0)))
    zeros_np = jnp.zeros((NP, H), jnp.float32)

    degp = _deg_call(dst)
    dinv, g = _g1_call(degp, x_pad, W1)

    for W_next, b_cur in ((W2, b1), (W3, b2), (Wf, b3)):
        p = _seg_call(g, src, dst, zeros_np)
        g = _comb_call(p, g, dinv, b_cur.reshape(1, H), W_next)

    p = _seg_call(g, src, dst, zeros_np)
    A, B = _post_call(
        p, g, dinv, bf.reshape(1, H), Wa1, ba1.reshape(1, H // 2),
        Wa2.reshape(1, H // 2), ba2.reshape(1, 1),
        Wc1[:H], Wc1[H:2 * H])

    S = _cls_call(A, B, src, dst)

    ea_pad = jnp.pad(edge_attr, ((0, 0), (0, 4)))
    We1p = jnp.pad(We1, ((0, 4), (0, 0)))
    return _edge_call(
        S, ea_pad, We1p, be1.reshape(1, H), We2, be2.reshape(1, H),
        Wc1[2 * H:], bc1.reshape(1, H), Wc2, bc2.reshape(1, H // 2),
        Wc3, bc3.reshape(1, 2))


# speedup vs baseline: 10.0888x; 1.4056x over previous
import functools

import jax
import jax.numpy as jnp
from jax import lax
from jax.experimental import pallas as pl
from jax.experimental.pallas import tpu as pltpu
from jax.experimental.pallas import tpu_sc as plsc

N = 10000
NP = 10240
E = 320000
H = 128
NC, NS = 2, 16
NW = NC * NS
C = 80
NCH = E // C
NCHW = NCH // NW
CD = 128
NCHD = E // CD
BN = 1024
BE = 2000

_mesh = plsc.VectorSubcoreMesh(
    core_axis_name="c", subcore_axis_name="s", num_cores=NC, num_subcores=NS)
_sc_params = pltpu.CompilerParams(needs_layout_passes=False)


def _wid():
    return lax.axis_index("c") * NS + lax.axis_index("s")


def _deg_body(dst_hbm, out_hbm, deg_v, idx_v):
    wid = _wid()
    nch_w = (NCHD - wid + NW - 1) // NW

    def zero(i, carry):
        deg_v[pl.ds(i * 16, 16)] = jnp.zeros((16,), jnp.float32)
        return carry
    lax.fori_loop(0, NP // 16, zero, 0)

    ones = jnp.ones((16,), jnp.float32)

    def chunk(i, carry):
        ch = wid + i * NW
        pltpu.sync_copy(dst_hbm.at[pl.ds(ch * CD, CD)], idx_v)

        def sub(j, c2):
            idx = idx_v[pl.ds(j * 16, 16)]
            plsc.addupdate_scatter(deg_v, [idx], ones)
            return c2
        lax.fori_loop(0, CD // 16, sub, 0)
        return carry
    lax.fori_loop(0, nch_w, chunk, 0)
    pltpu.sync_copy(deg_v, out_hbm.at[wid])


_deg_call = pl.kernel(
    _deg_body,
    out_type=jax.ShapeDtypeStruct((NW, NP), jnp.float32),
    mesh=_mesh,
    compiler_params=_sc_params,
    scratch_types=[
        pltpu.VMEM((NP,), jnp.float32),
        pltpu.VMEM((CD,), jnp.int32),
    ],
)


def _seg_body(g_hbm, src_hbm, dst_hbm, zero_hbm, p_hbm,
              idx_s, idx_d, rows, acc, gsem):
    cid = lax.axis_index("c")
    sid = lax.axis_index("s")
    wid = cid * NS + sid
    rpw = NP // NS

    pltpu.sync_copy(zero_hbm.at[pl.ds(sid * rpw, rpw)],
                    acc.at[pl.ds(sid * rpw, rpw)])
    plsc.subcore_barrier()

    def load_and_fire(i, b):
        ch = wid + i * NW
        pltpu.sync_copy(src_hbm.at[pl.ds(ch * C, C)], idx_s.at[b])
        pltpu.sync_copy(dst_hbm.at[pl.ds(ch * C, C)], idx_d.at[b])
        pltpu.async_copy(g_hbm.at[idx_s.at[b]], rows.at[b], gsem)

    def drain_and_scatter(b):
        pltpu.make_async_copy(g_hbm.at[idx_s.at[b]], rows.at[b], gsem).wait()
        pltpu.sync_copy(rows.at[b], acc.at[idx_d.at[b]], add=True)

    load_and_fire(0, 0)
    load_and_fire(1, 1)

    def pair(k, carry):
        for b in (0, 1):
            i = 2 * k + b
            drain_and_scatter(b)

            @pl.when(i + 2 < NCHW)
            def _():
                load_and_fire(i + 2, b)
        return carry
    lax.fori_loop(0, NCHW // 2, pair, 0)
    if NCHW % 2:
        drain_and_scatter(0)

    plsc.subcore_barrier()
    pltpu.sync_copy(acc.at[pl.ds(sid * rpw, rpw)],
                    p_hbm.at[cid].at[pl.ds(sid * rpw, rpw)])


_seg_call = pl.kernel(
    _seg_body,
    out_type=jax.ShapeDtypeStruct((NC, NP, H), jnp.float32),
    mesh=_mesh,
    compiler_params=_sc_params,
    scratch_types=[
        pltpu.VMEM((2, C), jnp.int32),
        pltpu.VMEM((2, C), jnp.int32),
        pltpu.VMEM((2, C, H), jnp.float32),
        pltpu.VMEM_SHARED((NP, H), jnp.float32),
        pltpu.SemaphoreType.DMA,
    ],
)


def _cls_body(a_hbm, b_hbm, row_hbm, col_hbm, s_hbm,
              idx_r, idx_c, rows_a, rows_b, sem):
    wid = _wid()

    def load_and_fire(i, b):
        ch = wid + i * NW
        pltpu.sync_copy(row_hbm.at[pl.ds(ch * C, C)], idx_r.at[b])
        pltpu.sync_copy(col_hbm.at[pl.ds(ch * C, C)], idx_c.at[b])
        pltpu.async_copy(a_hbm.at[idx_r.at[b]], rows_a.at[b], sem)
        pltpu.async_copy(b_hbm.at[idx_c.at[b]], rows_b.at[b], sem)

    def drain_add_store(i, b):
        ch = wid + i * NW
        pltpu.make_async_copy(a_hbm.at[idx_r.at[b]], rows_a.at[b], sem).wait()
        pltpu.make_async_copy(b_hbm.at[idx_c.at[b]], rows_b.at[b], sem).wait()

        def add_row(r, c2):
            for col in range(H // 16):
                sl = pl.ds(col * 16, 16)
                rows_a[b, r, sl] = rows_a[b, r, sl] + rows_b[b, r, sl]
            return c2
        lax.fori_loop(0, C, add_row, 0)
        pltpu.sync_copy(rows_a.at[b], s_hbm.at[pl.ds(ch * C, C)])

    load_and_fire(0, 0)
    load_and_fire(1, 1)

    def pair(k, carry):
        for b in (0, 1):
            i = 2 * k + b
            drain_add_store(i, b)

            @pl.when(i + 2 < NCHW)
            def _():
                load_and_fire(i + 2, b)
        return carry
    lax.fori_loop(0, NCHW // 2, pair, 0)
    if NCHW % 2:
        drain_add_store(NCHW - 1, 0)


_cls_call = pl.kernel(
    _cls_body,
    out_type=jax.ShapeDtypeStruct((E, H), jnp.float32),
    mesh=_mesh,
    compiler_params=_sc_params,
    scratch_types=[
        pltpu.VMEM((2, C), jnp.int32),
        pltpu.VMEM((2, C), jnp.int32),
        pltpu.VMEM((2, C, H), jnp.float32),
        pltpu.VMEM((2, C, H), jnp.float32),
        pltpu.SemaphoreType.DMA,
    ],
)


def _g1_kernel(degp_ref, x_ref, w_ref, dinv_ref, g_ref):
    deg = jnp.sum(degp_ref[...], axis=0) + 1.0
    dinv = lax.rsqrt(deg)
    dinv_ref[...] = dinv
    g_ref[...] = (x_ref[...] @ w_ref[...]) * dinv[:, None]


def _g1_call(degp, x_pad, W1):
    return pl.pallas_call(
        _g1_kernel,
        grid=(NP // BN,),
        in_specs=[
            pl.BlockSpec((NW, BN), lambda i: (0, i)),
            pl.BlockSpec((BN, H), lambda i: (i, 0)),
            pl.BlockSpec((H, H), lambda i: (0, 0)),
        ],
        out_specs=[
            pl.BlockSpec((BN,), lambda i: (i,)),
            pl.BlockSpec((BN, H), lambda i: (i, 0)),
        ],
        out_shape=[
            jax.ShapeDtypeStruct((NP,), jnp.float32),
            jax.ShapeDtypeStruct((NP, H), jnp.float32),
        ],
    )(degp, x_pad, W1)


def _comb_kernel(p_ref, g_ref, dinv_ref, b_ref, w_ref, out_ref):
    dinv = dinv_ref[...]
    h = jnp.maximum(
        (p_ref[0] + p_ref[1] + g_ref[...]) * dinv[:, None] + b_ref[...], 0.0)
    out_ref[...] = (h @ w_ref[...]) * dinv[:, None]


def _comb_call(p, g, dinv, b_row, W_next):
    return pl.pallas_call(
        _comb_kernel,
        grid=(NP // BN,),
        in_specs=[
            pl.BlockSpec((NC, BN, H), lambda i: (0, i, 0)),
            pl.BlockSpec((BN, H), lambda i: (i, 0)),
            pl.BlockSpec((BN,), lambda i: (i,)),
            pl.BlockSpec((1, H), lambda i: (0, 0)),
            pl.BlockSpec((H, H), lambda i: (0, 0)),
        ],
        out_specs=pl.BlockSpec((BN, H), lambda i: (i, 0)),
        out_shape=jax.ShapeDtypeStruct((NP, H), jnp.float32),
    )(p, g, dinv, b_row, W_next)


def _post_kernel(p_ref, g_ref, dinv_ref, bf_ref, wa1_ref, ba1_ref,
                 wa2_ref, ba2_ref, w1a_ref, w1b_ref, a_ref, b_out_ref):
    dinv = dinv_ref[...]
    h = jnp.maximum(
        (p_ref[0] + p_ref[1] + g_ref[...]) * dinv[:, None] + bf_ref[...], 0.0)
    t = jnp.maximum(h @ wa1_ref[...] + ba1_ref[...], 0.0)
    att = jax.nn.sigmoid(
        jnp.sum(t * wa2_ref[...], axis=1, keepdims=True) + ba2_ref[...])
    hg = h * att
    a_ref[...] = hg @ w1a_ref[...]
    b_out_ref[...] = hg @ w1b_ref[...]


def _post_call(p, g, dinv, bf_row, Wa1, ba1_row, wa2_row, ba2_11, W1a, W1b):
    return pl.pallas_call(
        _post_kernel,
        grid=(NP // BN,),
        in_specs=[
            pl.BlockSpec((NC, BN, H), lambda i: (0, i, 0)),
            pl.BlockSpec((BN, H), lambda i: (i, 0)),
            pl.BlockSpec((BN,), lambda i: (i,)),
            pl.BlockSpec((1, H), lambda i: (0, 0)),
            pl.BlockSpec((H, H // 2), lambda i: (0, 0)),
            pl.BlockSpec((1, H // 2), lambda i: (0, 0)),
            pl.BlockSpec((1, H // 2), lambda i: (0, 0)),
            pl.BlockSpec((1, 1), lambda i: (0, 0)),
            pl.BlockSpec((H, H), lambda i: (0, 0)),
            pl.BlockSpec((H, H), lambda i: (0, 0)),
        ],
        out_specs=[
            pl.BlockSpec((BN, H), lambda i: (i, 0)),
            pl.BlockSpec((BN, H), lambda i: (i, 0)),
        ],
        out_shape=[
            jax.ShapeDtypeStruct((NP, H), jnp.float32),
            jax.ShapeDtypeStruct((NP, H), jnp.float32),
        ],
    )(p, g, dinv, bf_row, Wa1, ba1_row, wa2_row, ba2_11, W1a, W1b)


def _edge_kernel(s_ref, ea_ref, we1_ref, be1_ref, we2_ref, be2_ref,
                 wc1c_ref, bc1_ref, wc2_ref, bc2_ref, wc3_ref, bc3_ref,
                 out_ref):
    t = jnp.maximum(ea_ref[...] @ we1_ref[...] + be1_ref[...], 0.0)
    wprime = we2_ref[...] @ wc1c_ref[...]
    cprime = be2_ref[...] @ wc1c_ref[...] + bc1_ref[...]
    z = jnp.maximum(t @ wprime + s_ref[...] + cprime, 0.0)
    z2 = jnp.maximum(z @ wc2_ref[...] + bc2_ref[...], 0.0)
    logits = z2 @ wc3_ref[...] + bc3_ref[...]
    m = jnp.max(logits, axis=1, keepdims=True)
    lse = m + jnp.log(jnp.sum(jnp.exp(logits - m), axis=1, keepdims=True))
    out_ref[...] = logits - lse


def _edge_call(S, ea_pad, We1p, be1_row, We2, be2_row, Wc1c, bc1_row,
               Wc2, bc2_row, Wc3, bc3_row):
    full = lambda shape: pl.BlockSpec(shape, lambda i: tuple(0 for _ in shape))
    return pl.pallas_call(
        _edge_kernel,
        grid=(E // BE,),
        in_specs=[
            pl.BlockSpec((BE, H), lambda i: (i, 0)),
            pl.BlockSpec((BE, 8), lambda i: (i, 0)),
            full((8, H)),
            full((1, H)),
            full((H, H)),
            full((1, H)),
            full((H, H)),
            full((1, H)),
            full((H, H // 2)),
            full((1, H // 2)),
            full((H // 2, 2)),
            full((1, 2)),
        ],
        out_specs=pl.BlockSpec((BE, 2), lambda i: (i, 0)),
        out_shape=jax.ShapeDtypeStruct((E, 2), jnp.float32),
    )(S, ea_pad, We1p, be1_row, We2, be2_row, Wc1c, bc1_row,
      Wc2, bc2_row, Wc3, bc3_row)


def kernel(x, edge_index, edge_attr, W1, b1, W2, b2, W3, b3, Wf, bf,
           We1, be1, We2, be2, Wa1, ba1, Wa2, ba2,
           Wc1, bc1, Wc2, bc2, Wc3, bc3):
    src = edge_index[0]
    dst = edge_index[1]
    x_pad = jnp.pad(x, ((0, NP - N), (0, 0)))
    zeros_np = jnp.zeros((NP, H), jnp.float32)

    degp = _deg_call(dst)
    dinv, g = _g1_call(degp, x_pad, W1)

    for W_next, b_cur in ((W2, b1), (W3, b2), (Wf, b3)):
        p = _seg_call(g, src, dst, zeros_np)
        g = _comb_call(p, g, dinv, b_cur.reshape(1, H), W_next)

    p = _seg_call(g, src, dst, zeros_np)
    A, B = _post_call(
        p, g, dinv, bf.reshape(1, H), Wa1, ba1.reshape(1, H // 2),
        Wa2.reshape(1, H // 2), ba2.reshape(1, 1),
        Wc1[:H], Wc1[H:2 * H])

    S = _cls_call(A, B, src, dst)

    ea_pad = jnp.pad(edge_attr, ((0, 0), (0, 4)))
    We1p = jnp.pad(We1, ((0, 4), (0, 0)))
    return _edge_call(
        S, ea_pad, We1p, be1.reshape(1, H), We2, be2.reshape(1, H),
        Wc1[2 * H:], bc1.reshape(1, H), Wc2, bc2.reshape(1, H // 2),
        Wc3, bc3.reshape(1, 2))
